# Initial kernel scaffold; baseline (speedup 1.0000x reference)
import jax
import jax.numpy as jnp
from jax.experimental import pallas as pl


def kernel(x_encode_0, x_encode_1, x_encode_2, x_encode_3, text_feature, params):
    raise NotImplementedError("write your pallas kernel here")


# speedup vs baseline: 2.1692x; 2.1692x over previous
import jax
import jax.numpy as jnp
from jax.experimental import pallas as pl
from jax.experimental.pallas import tpu as pltpu

_E = 8
_K = 2
_DIMS = [80, 64, 48, 32]
_OUTS = [64, 48, 32, 16]


def _gate_body(txt_ref, wg_ref, bg_ref, idx_ref, wts_ref, mi_ref):
    txt = txt_ref[...]
    mi_total = jnp.float32(0.0)
    for i in range(4):
        logits = jnp.dot(txt, wg_ref[i], preferred_element_type=jnp.float32)
        logits = logits + bg_ref[i][None, :]
        z = logits - jnp.max(logits, axis=-1, keepdims=True)
        ez = jnp.exp(z)
        probs = ez / jnp.sum(ez, axis=-1, keepdims=True)
        iota = jax.lax.broadcasted_iota(jnp.int32, probs.shape, 1)
        m0 = jnp.max(probs, axis=-1, keepdims=True)
        i0 = jnp.min(jnp.where(probs == m0, iota, _E), axis=-1, keepdims=True)
        sel0 = iota == i0
        probs1 = jnp.where(sel0, -jnp.inf, probs)
        m1 = jnp.max(probs1, axis=-1, keepdims=True)
        i1 = jnp.min(jnp.where(probs1 == m1, iota, _E), axis=-1, keepdims=True)
        sel1 = iota == i1
        s = m0 + m1
        idx_ref[i] = jnp.concatenate([i0, i1], axis=1)
        wts_ref[i] = jnp.concatenate([m0 / s, m1 / s], axis=1)
        importance = jnp.mean(probs, axis=0)
        load = jnp.mean((sel0 | sel1).astype(jnp.float32), axis=0)
        mi_total = mi_total + _E * jnp.sum(importance * load)
    mi_ref[...] = jnp.reshape(mi_total, (1, 1))


def _gating(txt, params):
    B = txt.shape[0]
    wg = jnp.stack([params['blk%d' % i]['Wg'] for i in range(4)])
    bg = jnp.stack([params['blk%d' % i]['bg'] for i in range(4)])
    idx, wts, mi = pl.pallas_call(
        _gate_body,
        out_shape=[
            jax.ShapeDtypeStruct((4, B, _K), jnp.int32),
            jax.ShapeDtypeStruct((4, B, _K), jnp.float32),
            jax.ShapeDtypeStruct((1, 1), jnp.float32),
        ],
    )(txt, wg, bg)
    return idx, wts, mi[0, 0]


def _moe(idx, wts, x, skip, p):
    B, d, N = x.shape
    TN = min(N, 4096)
    T = N // TN
    has_skip = skip is not None

    def body(idx_ref, wts_ref, x_ref, *rest):
        if has_skip:
            skip_ref, w1_ref, b1_ref, w2_ref, b2_ref, out_ref = rest
        else:
            w1_ref, b1_ref, w2_ref, b2_ref, out_ref = rest
            skip_ref = None
        b = pl.program_id(0)
        k = pl.program_id(2)
        xv = x_ref[0]
        if skip_ref is not None:
            xv = xv + skip_ref[0]
        h = jnp.dot(w1_ref[0], xv, preferred_element_type=jnp.float32)
        h = jax.nn.gelu(h + b1_ref[0])
        o = jnp.dot(w2_ref[0], h, preferred_element_type=jnp.float32)
        o = o + b2_ref[0]
        wt = wts_ref[b, k]

        @pl.when(k == 0)
        def _():
            out_ref[0] = xv + wt * o

        @pl.when(k == 1)
        def _():
            out_ref[0] = out_ref[0] + wt * o

    def bmap(b, t, k, idx_ref, wts_ref):
        return (b, 0, t)

    def emap(b, t, k, idx_ref, wts_ref):
        return (idx_ref[b, k], 0, 0)

    in_specs = [pl.BlockSpec((1, d, TN), bmap)]
    args = [x]
    if has_skip:
        in_specs.append(pl.BlockSpec((1, d, TN), bmap))
        args.append(skip)
    in_specs += [
        pl.BlockSpec((1, 2 * d, d), emap),
        pl.BlockSpec((1, 2 * d, 1), emap),
        pl.BlockSpec((1, d, 2 * d), emap),
        pl.BlockSpec((1, d, 1), emap),
    ]
    args += [
        p['W1'].transpose(0, 2, 1),
        p['b1'].reshape(_E, 2 * d, 1),
        p['W2'].transpose(0, 2, 1),
        p['b2'].reshape(_E, d, 1),
    ]
    return pl.pallas_call(
        body,
        grid_spec=pltpu.PrefetchScalarGridSpec(
            num_scalar_prefetch=2,
            grid=(B, T, _K),
            in_specs=in_specs,
            out_specs=pl.BlockSpec((1, d, TN), bmap),
        ),
        out_shape=jax.ShapeDtypeStruct((B, d, N), jnp.float32),
    )(idx, wts, *args)


def _shift_tokens(y, s):
    d, N = y.shape
    if s == 0:
        return y
    zeros = jnp.zeros((d, abs(s)), jnp.float32)
    if s > 0:
        return jnp.concatenate([y[:, s:], zeros], axis=1)
    return jnp.concatenate([zeros, y[:, :s]], axis=1)


def _conv(y, wu, bu, H, W, leaky):
    B, d, N = y.shape
    co = wu.shape[0]
    wtaps = wu.transpose(2, 3, 0, 1).reshape(9, co, d)
    bias = bu.reshape(1, co, 1)

    def body(y_ref, w_ref, b_ref, out_ref):
        yv = y_ref[0]
        n_iota = jax.lax.broadcasted_iota(jnp.int32, (1, N), 1)
        colv = n_iota % W
        rowv = n_iota // W
        acc = jnp.zeros((co, N), jnp.float32)
        for t in range(9):
            dy, dx = t // 3 - 1, t % 3 - 1
            ys = _shift_tokens(yv, dy * W + dx)
            ok = jnp.ones((1, N), jnp.bool_)
            if dy == -1:
                ok = rowv >= 1
            elif dy == 1:
                ok = rowv < H - 1
            if dx == -1:
                ok = ok & (colv >= 1)
            elif dx == 1:
                ok = ok & (colv < W - 1)
            if not (dy == 0 and dx == 0):
                ys = jnp.where(ok, ys, 0.0)
            acc = acc + jnp.dot(w_ref[t], ys,
                                preferred_element_type=jnp.float32)
        acc = acc + b_ref[0]
        if leaky:
            acc = jnp.where(acc >= 0, acc, 0.01 * acc)
        out_ref[0] = acc

    return pl.pallas_call(
        body,
        grid=(B,),
        in_specs=[
            pl.BlockSpec((1, d, N), lambda b: (b, 0, 0)),
            pl.BlockSpec((9, co, d), lambda b: (0, 0, 0)),
            pl.BlockSpec((1, co, 1), lambda b: (0, 0, 0)),
        ],
        out_specs=pl.BlockSpec((1, co, N), lambda b: (b, 0, 0)),
        out_shape=jax.ShapeDtypeStruct((B, co, N), jnp.float32),
    )(y, wtaps, bias)


def _pixel_shuffle(y, o, H, W):
    B = y.shape[0]
    y = y.reshape(B, o, 2, 2, H, W)
    y = y.transpose(0, 1, 4, 2, 5, 3)
    return y.reshape(B, o, 4 * H * W)


def kernel(x_encode_0, x_encode_1, x_encode_2, x_encode_3, text_feature,
           params):
    B = text_feature.shape[0]
    idx, wts, mi_total = _gating(text_feature, params)

    xs = [x_encode_3, x_encode_2, x_encode_1, x_encode_0]
    y_tok = None
    for i in range(4):
        p = params['blk%d' % i]
        d, o = _DIMS[i], _OUTS[i]
        _, _, H, W = xs[i].shape
        x_tok = xs[i].reshape(B, d, H * W)
        ymoe = _moe(idx[i], wts[i], x_tok, y_tok, p)
        yconv = _conv(ymoe, p['Wu'], p['bu'], H, W, leaky=True)
        y_tok = _pixel_shuffle(yconv, o, H, W)

    img = _conv(y_tok, params['Wc'], params['bc'], 256, 256, leaky=False)
    return img.reshape(B, 1, 256, 256), mi_total


# speedup vs baseline: 2.3600x; 1.0880x over previous
import jax
import jax.numpy as jnp
from jax.experimental import pallas as pl
from jax.experimental.pallas import tpu as pltpu

_E = 8
_K = 2
_DIMS = [80, 64, 48, 32]
_OUTS = [64, 48, 32, 16]
_HWS = [16, 32, 64, 128]


def _gate_body(txt_ref, wg_ref, bg_ref, idx_ref, wts_ref, mi_ref):
    txt = txt_ref[...]
    mi_total = jnp.float32(0.0)
    for i in range(4):
        logits = jnp.dot(txt, wg_ref[i], preferred_element_type=jnp.float32)
        logits = logits + bg_ref[i][None, :]
        z = logits - jnp.max(logits, axis=-1, keepdims=True)
        ez = jnp.exp(z)
        probs = ez / jnp.sum(ez, axis=-1, keepdims=True)
        iota = jax.lax.broadcasted_iota(jnp.int32, probs.shape, 1)
        m0 = jnp.max(probs, axis=-1, keepdims=True)
        i0 = jnp.min(jnp.where(probs == m0, iota, _E), axis=-1, keepdims=True)
        sel0 = iota == i0
        probs1 = jnp.where(sel0, -jnp.inf, probs)
        m1 = jnp.max(probs1, axis=-1, keepdims=True)
        i1 = jnp.min(jnp.where(probs1 == m1, iota, _E), axis=-1, keepdims=True)
        sel1 = iota == i1
        s = m0 + m1
        idx_ref[i] = jnp.concatenate([i0, i1], axis=1)
        wts_ref[i] = jnp.concatenate([m0 / s, m1 / s], axis=1)
        importance = jnp.mean(probs, axis=0)
        load = jnp.mean((sel0 | sel1).astype(jnp.float32), axis=0)
        mi_total = mi_total + _E * jnp.sum(importance * load)
    mi_ref[...] = jnp.reshape(mi_total, (1, 1))


def _gating(txt, params):
    B = txt.shape[0]
    wg = jnp.stack([params['blk%d' % i]['Wg'] for i in range(4)])
    bg = jnp.stack([params['blk%d' % i]['bg'] for i in range(4)])
    idx, wts, mi = pl.pallas_call(
        _gate_body,
        out_shape=[
            jax.ShapeDtypeStruct((4, B, _K), jnp.int32),
            jax.ShapeDtypeStruct((4, B, _K), jnp.float32),
            jax.ShapeDtypeStruct((1, 1), jnp.float32),
        ],
    )(txt, wg, bg)
    return idx, wts, mi[0, 0]


def _moe(idx, wts, x, skip, p):
    B, d, N = x.shape
    TN = min(N, 4096)
    T = N // TN
    has_skip = skip is not None

    def body(idx_ref, wts_ref, x_ref, *rest):
        if has_skip:
            skip_ref, w1_ref, b1_ref, w2_ref, b2_ref, out_ref = rest
        else:
            w1_ref, b1_ref, w2_ref, b2_ref, out_ref = rest
            skip_ref = None
        b = pl.program_id(0)
        e0 = idx_ref[b, 0]
        e1 = idx_ref[b, 1]
        wt0 = wts_ref[b, 0]
        wt1 = wts_ref[b, 1]
        w1c = jnp.concatenate([w1_ref[e0], w1_ref[e1]], axis=0)
        b1c = jnp.concatenate([b1_ref[e0], b1_ref[e1]], axis=0)
        w2m = jnp.concatenate([wt0 * w2_ref[e0], wt1 * w2_ref[e1]],
                              axis=1)
        b2m = wt0 * b2_ref[e0] + wt1 * b2_ref[e1]
        xv = x_ref[0]
        if skip_ref is not None:
            xv = xv + skip_ref[0]
        h = jnp.dot(w1c, xv, preferred_element_type=jnp.float32)
        h = jax.nn.gelu(h + b1c)
        ov = jnp.dot(w2m, h, preferred_element_type=jnp.float32)
        out_ref[0] = xv + ov + b2m

    def bmap(b, t, idx_ref, wts_ref):
        return (b, 0, t)

    def wmap(b, t, idx_ref, wts_ref):
        return (0, 0, 0)

    in_specs = [pl.BlockSpec((1, d, TN), bmap)]
    args = [x]
    if has_skip:
        in_specs.append(pl.BlockSpec((1, d, TN), bmap))
        args.append(skip)
    in_specs += [
        pl.BlockSpec((_E, 2 * d, d), wmap),
        pl.BlockSpec((_E, 2 * d, 1), wmap),
        pl.BlockSpec((_E, d, 2 * d), wmap),
        pl.BlockSpec((_E, d, 1), wmap),
    ]
    args += [
        p['W1'].transpose(0, 2, 1),
        p['b1'].reshape(_E, 2 * d, 1),
        p['W2'].transpose(0, 2, 1),
        p['b2'].reshape(_E, d, 1),
    ]
    return pl.pallas_call(
        body,
        grid_spec=pltpu.PrefetchScalarGridSpec(
            num_scalar_prefetch=2,
            grid=(B, T),
            in_specs=in_specs,
            out_specs=pl.BlockSpec((1, d, TN), bmap),
        ),
        out_shape=jax.ShapeDtypeStruct((B, d, N), jnp.float32),
    )(idx, wts, *args)


def _shift_tokens(y, s):
    d, N = y.shape
    if s == 0:
        return y
    zeros = jnp.zeros((d, abs(s)), jnp.float32)
    if s > 0:
        return jnp.concatenate([y[:, s:], zeros], axis=1)
    return jnp.concatenate([zeros, y[:, :s]], axis=1)


def _conv(y, wu, bu, H, W, leaky):
    B, d, N = y.shape
    co = wu.shape[0]
    wflat = wu.transpose(0, 2, 3, 1).reshape(co, 9 * d)
    bias = bu.reshape(1, co, 1)

    def body(y_ref, w_ref, b_ref, out_ref):
        yv = y_ref[0]
        n_iota = jax.lax.broadcasted_iota(jnp.int32, (1, N), 1)
        colv = n_iota % W
        rowv = n_iota // W
        taps = []
        for t in range(9):
            dy, dx = t // 3 - 1, t % 3 - 1
            ys = _shift_tokens(yv, dy * W + dx)
            ok = None
            if dy == -1:
                ok = rowv >= 1
            elif dy == 1:
                ok = rowv < H - 1
            if dx == -1:
                ok = (colv >= 1) if ok is None else (ok & (colv >= 1))
            elif dx == 1:
                ok = (colv < W - 1) if ok is None else (ok & (colv < W - 1))
            if ok is not None:
                ys = jnp.where(ok, ys, 0.0)
            taps.append(ys)
        ystack = jnp.concatenate(taps, axis=0)
        acc = jnp.dot(w_ref[...], ystack, preferred_element_type=jnp.float32)
        acc = acc + b_ref[0]
        if leaky:
            acc = jnp.where(acc >= 0, acc, 0.01 * acc)
        out_ref[0] = acc

    return pl.pallas_call(
        body,
        grid=(B,),
        in_specs=[
            pl.BlockSpec((1, d, N), lambda b: (b, 0, 0)),
            pl.BlockSpec((co, 9 * d), lambda b: (0, 0)),
            pl.BlockSpec((1, co, 1), lambda b: (0, 0, 0)),
        ],
        out_specs=pl.BlockSpec((1, co, N), lambda b: (b, 0, 0)),
        out_shape=jax.ShapeDtypeStruct((B, co, N), jnp.float32),
    )(y, wflat, bias)


def _pixel_shuffle(y, o, H, W):
    B = y.shape[0]
    y = y.reshape(B, o, 2, 2, H, W)
    y = y.transpose(0, 1, 4, 2, 5, 3)
    return y.reshape(B, o, 4 * H * W)


def kernel(x_encode_0, x_encode_1, x_encode_2, x_encode_3, text_feature,
           params):
    B = text_feature.shape[0]
    idx, wts, mi_total = _gating(text_feature, params)

    xs = [x_encode_3, x_encode_2, x_encode_1, x_encode_0]
    y_tok = None
    for i in range(4):
        p = params['blk%d' % i]
        d, o, HW = _DIMS[i], _OUTS[i], _HWS[i]
        x_tok = xs[i].reshape(B, d, HW * HW)
        ymoe = _moe(idx[i], wts[i], x_tok, y_tok, p)
        yconv = _conv(ymoe, p['Wu'], p['bu'], HW, HW, leaky=True)
        y_tok = _pixel_shuffle(yconv, o, HW, HW)

    img = _conv(y_tok, params['Wc'], params['bc'], 256, 256, leaky=False)
    return img.reshape(B, 1, 256, 256), mi_total
